# baseline (speedup 1.0000x reference)
import functools

import jax
import jax.numpy as jnp
from jax import lax
from jax.experimental import pallas as pl
from jax.experimental.pallas import tpu as pltpu
from jax.experimental.pallas import tpu_sc as plsc

_N_NODES = 100000
_MAX_DEG = 64
_PADW = 128
_BATCH = 16384
_NUM_SAMPLES = 16

_info = plsc.get_sparse_core_info()
_NC, _NS, _L = _info.num_cores, _info.num_subcores, _info.num_lanes
_NW = _NC * _NS
_B_PER_W = _BATCH // _NW


def _make_sampler():
    mesh = plsc.VectorSubcoreMesh(core_axis_name="c", subcore_axis_name="s")

    @functools.partial(
        pl.kernel,
        mesh=mesh,
        compiler_params=pltpu.CompilerParams(
            skip_device_barrier=True,
        ),
        out_type=jax.ShapeDtypeStruct((_BATCH * _NUM_SAMPLES,), jnp.int32),
        scratch_types=[
            pltpu.VMEM((_B_PER_W,), jnp.int32),
            pltpu.VMEM((_L,), jnp.int32),
            pltpu.VMEM((_B_PER_W, _PADW), jnp.int32),
            pltpu.VMEM((_B_PER_W * _NUM_SAMPLES,), jnp.int32),
            pltpu.SemaphoreType.DMA,
        ],
    )
    def sampler(ids_hbm, adj_hbm, cols_hbm, out_hbm,
                ids_v, cols_v, rows_v, out_v, sem):
        wid = lax.axis_index("s") * _NC + lax.axis_index("c")
        base = wid * _B_PER_W
        pltpu.sync_copy(ids_hbm.at[pl.ds(base, _B_PER_W)], ids_v)
        pltpu.sync_copy(cols_hbm, cols_v)
        pltpu.async_copy(adj_hbm.at[ids_v], rows_v, sem).wait()

        cols = cols_v[...]
        lane = jnp.bitwise_and(cols, _L - 1)
        sel0 = cols < _L
        sel1 = cols < 2 * _L
        sel2 = cols < 3 * _L

        def body(i, carry):
            q0 = rows_v[i, pl.ds(0 * _L, _L)]
            q1 = rows_v[i, pl.ds(1 * _L, _L)]
            q2 = rows_v[i, pl.ds(2 * _L, _L)]
            q3 = rows_v[i, pl.ds(3 * _L, _L)]
            g0 = q0.at[lane].get(mode="promise_in_bounds")
            g1 = q1.at[lane].get(mode="promise_in_bounds")
            g2 = q2.at[lane].get(mode="promise_in_bounds")
            g3 = q3.at[lane].get(mode="promise_in_bounds")
            r01 = jnp.where(sel0, g0, g1)
            r23 = jnp.where(sel2, g2, g3)
            out_v[pl.ds(i * _NUM_SAMPLES, _NUM_SAMPLES)] = jnp.where(
                sel1, r01, r23)
            return carry

        lax.fori_loop(0, _B_PER_W, body, 0)
        pltpu.sync_copy(
            out_v,
            out_hbm.at[pl.ds(base * _NUM_SAMPLES, _B_PER_W * _NUM_SAMPLES)])

    return sampler


_sampler = _make_sampler()


def kernel(ids, adj_info, num_samples):
    perm = jax.random.permutation(jax.random.key(42), _MAX_DEG)
    col_idx = jnp.minimum(jnp.arange(_NUM_SAMPLES), num_samples - 1)
    cols = perm[col_idx].astype(jnp.int32)
    adj_pad = jnp.pad(adj_info, ((0, 0), (0, _PADW - _MAX_DEG)))
    flat = _sampler(ids, adj_pad, cols)
    return flat.reshape(_BATCH, _NUM_SAMPLES)


# speedup vs baseline: 1.1474x; 1.1474x over previous
import functools

import jax
import jax.numpy as jnp
from jax import lax
from jax.experimental import pallas as pl
from jax.experimental.pallas import tpu as pltpu
from jax.experimental.pallas import tpu_sc as plsc

_N_NODES = 100000
_MAX_DEG = 64
_BATCH = 16384
_NUM_SAMPLES = 16

_info = plsc.get_sparse_core_info()
_NC, _NS, _L = _info.num_cores, _info.num_subcores, _info.num_lanes
_NW = _NC * _NS
_B_PER_W = _BATCH // _NW


def _make_sampler():
    mesh = plsc.VectorSubcoreMesh(core_axis_name="c", subcore_axis_name="s")

    @functools.partial(
        pl.kernel,
        mesh=mesh,
        compiler_params=pltpu.CompilerParams(
            skip_device_barrier=True,
            disable_semaphore_checks=True,
            disable_bounds_checks=True,
        ),
        out_type=jax.ShapeDtypeStruct((_BATCH * _NUM_SAMPLES,), jnp.int32),
        scratch_types=[
            pltpu.VMEM((_B_PER_W,), jnp.int32),
            pltpu.VMEM((_L,), jnp.int32),
            pltpu.VMEM((_B_PER_W, _MAX_DEG), jnp.int32),
            pltpu.VMEM((_B_PER_W * _NUM_SAMPLES,), jnp.int32),
            pltpu.SemaphoreType.DMA,
        ],
    )
    def sampler(ids_hbm, adj_hbm, cols_hbm, out_hbm,
                ids_v, cols_v, rows_v, out_v, sem):
        wid = lax.axis_index("s") * _NC + lax.axis_index("c")
        base = wid * _B_PER_W
        pltpu.sync_copy(ids_hbm.at[pl.ds(base, _B_PER_W)], ids_v)
        pltpu.sync_copy(cols_hbm, cols_v)

        def fire(c, carry):
            v = ids_v[pl.ds(c * _L, _L)]
            for k in range(_L):
                pltpu.async_copy(adj_hbm.at[v[k]], rows_v.at[c * _L + k], sem)
            return carry

        lax.fori_loop(0, _B_PER_W // _L, fire, 0)
        pltpu.make_async_copy(
            adj_hbm.at[pl.ds(0, _B_PER_W)], rows_v, sem).wait()

        cols = cols_v[...]
        lane = jnp.bitwise_and(cols, _L - 1)
        sel0 = cols < _L
        sel1 = cols < 2 * _L
        sel2 = cols < 3 * _L

        def body(i, carry):
            q0 = rows_v[i, pl.ds(0 * _L, _L)]
            q1 = rows_v[i, pl.ds(1 * _L, _L)]
            q2 = rows_v[i, pl.ds(2 * _L, _L)]
            q3 = rows_v[i, pl.ds(3 * _L, _L)]
            g0 = q0.at[lane].get(mode="promise_in_bounds")
            g1 = q1.at[lane].get(mode="promise_in_bounds")
            g2 = q2.at[lane].get(mode="promise_in_bounds")
            g3 = q3.at[lane].get(mode="promise_in_bounds")
            r01 = jnp.where(sel0, g0, g1)
            r23 = jnp.where(sel2, g2, g3)
            out_v[pl.ds(i * _NUM_SAMPLES, _NUM_SAMPLES)] = jnp.where(
                sel1, r01, r23)
            return carry

        lax.fori_loop(0, _B_PER_W, body, 0)
        pltpu.sync_copy(
            out_v,
            out_hbm.at[pl.ds(base * _NUM_SAMPLES, _B_PER_W * _NUM_SAMPLES)])

    return sampler


_sampler = _make_sampler()


def kernel(ids, adj_info, num_samples):
    perm = jax.random.permutation(jax.random.key(42), _MAX_DEG)
    col_idx = jnp.minimum(jnp.arange(_NUM_SAMPLES), num_samples - 1)
    cols = perm[col_idx].astype(jnp.int32)
    flat = _sampler(ids, adj_info, cols)
    return flat.reshape(_BATCH, _NUM_SAMPLES)
